# Initial kernel scaffold; baseline (speedup 1.0000x reference)
import jax
import jax.numpy as jnp
from jax.experimental import pallas as pl


def kernel(gatEmb, odNum, path_idx, od_idx):
    raise NotImplementedError("write your pallas kernel here")


# speedup vs baseline: 16.0988x; 16.0988x over previous
import jax
import jax.numpy as jnp
from jax import lax
from jax.experimental import pallas as pl
from jax.experimental.pallas import tpu as pltpu
from jax.experimental.pallas import tpu_sc as plsc

N_PATH = 10000
N_OD = 10000
E = 320000
D = 128
L = 16
CHUNK = 128
N_CHUNKS = E // CHUNK
NC = 2
NS = 16
NW = NC * NS
SEG_PAD = 10240
ROWS_PER_TILE = 624


def _sc_body(gat_hbm, odnum_hbm, path_hbm, od_hbm, out_hbm,
             gat_v, seg_v, pchunk, ochunk, prob_v, rows_v, zseg_v,
             sem, seg_sh, acc_sh):
    c = lax.axis_index("c")
    s = lax.axis_index("s")
    w = s * NC + c

    pltpu.sync_copy(gat_hbm, gat_v)

    def _mx(k, m):
        return jnp.maximum(m, gat_v[pl.ds(k * L, L)])

    m = lax.fori_loop(0, N_PATH // L, _mx,
                      jnp.full((L,), -1e30, dtype=jnp.float32))
    lanes = lax.iota(jnp.int32, L)
    for dstep in (1, 2, 4, 8):
        m = jnp.maximum(m, m.at[lanes ^ dstep].get(mode="promise_in_bounds"))
    g = m

    def _zrows(i, _):
        for j in range(D // L):
            rows_v[i, pl.ds(j * L, L)] = jnp.zeros((L,), jnp.float32)
        return 0

    lax.fori_loop(0, CHUNK, _zrows, 0)

    def _zseg(k, _):
        zseg_v[pl.ds(k * L, L)] = jnp.zeros((L,), jnp.float32)
        return 0

    lax.fori_loop(0, (SEG_PAD // NS) // L, _zseg, 0)
    pltpu.sync_copy(zseg_v, seg_sh.at[pl.ds(s * (SEG_PAD // NS), SEG_PAD // NS)])
    for r in range(4):
        pltpu.sync_copy(rows_v,
                        acc_sh.at[pl.ds(s * ROWS_PER_TILE + r * CHUNK, CHUNK), :])
    pltpu.sync_copy(rows_v.at[pl.ds(0, 112), :],
                    acc_sh.at[pl.ds(s * ROWS_PER_TILE + 4 * CHUNK, 112), :])

    @pl.when(s == 0)
    def _():
        pltpu.sync_copy(rows_v.at[pl.ds(0, L), :],
                        acc_sh.at[pl.ds(NS * ROWS_PER_TILE, L), :])

    plsc.subcore_barrier()
    plsc.subcore_barrier()

    na = jnp.where(s < N_CHUNKS % NS, N_CHUNKS // NS + 1, N_CHUNKS // NS)

    def _phase_a(k, _):
        base = pl.multiple_of((s + NS * k) * CHUNK, CHUNK)
        pltpu.sync_copy(path_hbm.at[pl.ds(base, CHUNK)], pchunk)
        pltpu.sync_copy(od_hbm.at[pl.ds(base, CHUNK)], ochunk)
        for j in range(CHUNK // L):
            idxv = pchunk[pl.ds(j * L, L)]
            he = plsc.load_gather(gat_v, [idxv])
            prob_v[pl.ds(j * L, L)] = jnp.exp(he - g)
        pltpu.sync_copy(prob_v, seg_sh.at[ochunk], add=True)
        return 0

    lax.fori_loop(0, na, _phase_a, 0)
    plsc.subcore_barrier()
    plsc.subcore_barrier()

    pltpu.sync_copy(seg_sh, seg_v)

    nb = jnp.where(w < N_CHUNKS % NW, N_CHUNKS // NW + 1, N_CHUNKS // NW)

    def _phase_b(k, _):
        base = pl.multiple_of((w + NW * k) * CHUNK, CHUNK)
        pltpu.sync_copy(path_hbm.at[pl.ds(base, CHUNK)], pchunk)
        pltpu.sync_copy(od_hbm.at[pl.ds(base, CHUNK)], ochunk)
        cp = pltpu.async_copy(odnum_hbm.at[ochunk], rows_v, sem)
        for j in range(CHUNK // L):
            pidx = pchunk[pl.ds(j * L, L)]
            oidx = ochunk[pl.ds(j * L, L)]
            he = plsc.load_gather(gat_v, [pidx])
            ssum = plsc.load_gather(seg_v, [oidx])
            prob_v[pl.ds(j * L, L)] = jnp.exp(he - g) / ssum
        cp.wait()

        def _scale(i, _):
            p = plsc.load_gather(prob_v, [jnp.full((L,), i, jnp.int32)])
            for j in range(D // L):
                rows_v[i, pl.ds(j * L, L)] = rows_v[i, pl.ds(j * L, L)] * p
            return 0

        lax.fori_loop(0, CHUNK, _scale, 0)
        pltpu.sync_copy(rows_v, acc_sh.at[pchunk], add=True)
        return 0

    lax.fori_loop(0, nb, _phase_b, 0)
    plsc.subcore_barrier()
    plsc.subcore_barrier()

    r0 = s * ROWS_PER_TILE
    pltpu.sync_copy(acc_sh.at[pl.ds(r0, ROWS_PER_TILE), :],
                    out_hbm.at[c, pl.ds(r0, ROWS_PER_TILE), :])

    @pl.when(s == 0)
    def _():
        pltpu.sync_copy(acc_sh.at[pl.ds(NS * ROWS_PER_TILE, L), :],
                        out_hbm.at[c, pl.ds(NS * ROWS_PER_TILE, L), :])


@jax.jit
def _sc_call(gat, odnum, path_idx, od_idx):
    mesh = plsc.VectorSubcoreMesh(core_axis_name="c", subcore_axis_name="s")
    kfn = pl.kernel(
        _sc_body,
        mesh=mesh,
        compiler_params=pltpu.CompilerParams(needs_layout_passes=False),
        out_type=jax.ShapeDtypeStruct((NC, N_PATH, D), jnp.float32),
        scratch_types=[
            pltpu.VMEM((N_PATH,), jnp.float32),
            pltpu.VMEM((SEG_PAD,), jnp.float32),
            pltpu.VMEM((CHUNK,), jnp.int32),
            pltpu.VMEM((CHUNK,), jnp.int32),
            pltpu.VMEM((CHUNK,), jnp.float32),
            pltpu.VMEM((CHUNK, D), jnp.float32),
            pltpu.VMEM((SEG_PAD // NS,), jnp.float32),
            pltpu.SemaphoreType.DMA,
            pltpu.VMEM_SHARED((SEG_PAD,), jnp.float32),
            pltpu.VMEM_SHARED((N_PATH, D), jnp.float32),
        ],
    )
    return kfn(gat, odnum, path_idx, od_idx)


def _add_body(p_ref, o_ref):
    o_ref[...] = p_ref[0] + p_ref[1]


@jax.jit
def _combine(partials):
    return pl.pallas_call(
        _add_body,
        out_shape=jax.ShapeDtypeStruct((N_PATH, D), jnp.float32),
        grid=(10,),
        in_specs=[pl.BlockSpec((NC, N_PATH // 10, D), lambda i: (0, i, 0))],
        out_specs=pl.BlockSpec((N_PATH // 10, D), lambda i: (i, 0)),
    )(partials)


def kernel(gatEmb, odNum, path_idx, od_idx):
    gat = jnp.reshape(gatEmb, (N_PATH,))
    partials = _sc_call(gat, odNum, path_idx, od_idx)
    return _combine(partials)


# speedup vs baseline: 31.6182x; 1.9640x over previous
import jax
import jax.numpy as jnp
from jax import lax
from jax.experimental import pallas as pl
from jax.experimental.pallas import tpu as pltpu
from jax.experimental.pallas import tpu_sc as plsc

N_PATH = 10000
N_OD = 10000
E = 320000
D = 128
L = 16
CHUNK = 96
NC = 2
NS = 16
NW = NC * NS
SEG_PAD = 10240
ROWS_PER_TILE = 624

EPT = E // NS
A_FULL = EPT // CHUNK
A_TAIL = EPT - A_FULL * CHUNK
EPW = E // NW
B_FULL = EPW // CHUNK
B_TAIL = EPW - B_FULL * CHUNK


def _sc_body(gat_hbm, odnum_hbm, path_hbm, od_hbm, out_hbm,
             gat_v, seg_v, pidx2, oidx2, scidx2, ex2, rows2,
             tidxa, tidxb, sem_i, sem_g, sem_s, seg_sh, acc_sh):
    c = lax.axis_index("c")
    s = lax.axis_index("s")
    w = s * NC + c

    pltpu.sync_copy(gat_hbm, gat_v)

    def _mx(k, m):
        return jnp.maximum(m, gat_v[pl.ds(k * L, L)])

    m = lax.fori_loop(0, N_PATH // L, _mx,
                      jnp.full((L,), -1e30, dtype=jnp.float32))
    lanes = lax.iota(jnp.int32, L)
    for dstep in (1, 2, 4, 8):
        m = jnp.maximum(m, m.at[lanes ^ dstep].get(mode="promise_in_bounds"))
    g = m

    def _zrows(i, _):
        for j in range(D // L):
            rows2[0, i, pl.ds(j * L, L)] = jnp.zeros((L,), jnp.float32)
        return 0

    lax.fori_loop(0, CHUNK, _zrows, 0)

    for r in range(5):
        pltpu.sync_copy(rows2.at[0, 0, :],
                        seg_sh.at[pl.ds(s * (SEG_PAD // NS) + r * D, D)])
    for r in range(6):
        pltpu.sync_copy(rows2.at[0],
                        acc_sh.at[pl.ds(s * ROWS_PER_TILE + r * CHUNK, CHUNK), :])
    pltpu.sync_copy(rows2.at[0, pl.ds(0, 48), :],
                    acc_sh.at[pl.ds(s * ROWS_PER_TILE + 6 * CHUNK, 48), :])

    @pl.when(s == 0)
    def _():
        pltpu.sync_copy(rows2.at[0, pl.ds(0, L), :],
                        acc_sh.at[pl.ds(NS * ROWS_PER_TILE, L), :])

    plsc.subcore_barrier()
    plsc.subcore_barrier()

    a_base = s * EPT

    def _idx_load(b, base, n):
        pltpu.async_copy(path_hbm.at[pl.ds(base, n)], pidx2.at[b], sem_i.at[b])
        pltpu.async_copy(od_hbm.at[pl.ds(base, n)], oidx2.at[b], sem_i.at[b])

    def _idx_wait(b):
        pltpu.make_async_copy(path_hbm.at[pl.ds(0, CHUNK)], pidx2.at[b],
                              sem_i.at[b]).wait()
        pltpu.make_async_copy(od_hbm.at[pl.ds(0, CHUNK)], oidx2.at[b],
                              sem_i.at[b]).wait()

    def _seg_scatter_wait(b):
        pltpu.make_async_copy(ex2.at[b], seg_sh.at[scidx2.at[b]],
                              sem_s.at[b]).wait()

    for b in range(2):
        _idx_load(b, pl.multiple_of(a_base + b * CHUNK, 32), CHUNK)

    def _phase_a(gidx, _):
        for b in range(2):
            k = 2 * gidx + b

            @pl.when(k >= 2)
            def _():
                _seg_scatter_wait(b)

            _idx_wait(b)
            for j in range(CHUNK // L):
                he = plsc.load_gather(gat_v, [pidx2[b, pl.ds(j * L, L)]])
                ex2[b, pl.ds(j * L, L)] = jnp.exp(he - g)
            for j in range(CHUNK // L):
                scidx2[b, pl.ds(j * L, L)] = oidx2[b, pl.ds(j * L, L)]
            nxt = jnp.minimum(k + 2, A_FULL - 1)
            _idx_load(b, pl.multiple_of(a_base + nxt * CHUNK, 32), CHUNK)
            pltpu.async_copy(ex2.at[b], seg_sh.at[scidx2.at[b]], sem_s.at[b],
                             add=True)
        return 0

    lax.fori_loop(0, A_FULL // 2, _phase_a, 0)
    for b in range(2):
        _idx_wait(b)
        _seg_scatter_wait(b)

    t_base = pl.multiple_of(a_base + A_FULL * CHUNK, 32)
    pltpu.sync_copy(path_hbm.at[pl.ds(t_base, A_TAIL)],
                    pidx2.at[0, pl.ds(0, A_TAIL)])
    pltpu.sync_copy(od_hbm.at[pl.ds(t_base, A_TAIL)], tidxa)
    for j in range(A_TAIL // L):
        he = plsc.load_gather(gat_v, [pidx2[0, pl.ds(j * L, L)]])
        ex2[0, pl.ds(j * L, L)] = jnp.exp(he - g)
    pltpu.sync_copy(ex2.at[0, pl.ds(0, A_TAIL)], seg_sh.at[tidxa], add=True)

    plsc.subcore_barrier()
    plsc.subcore_barrier()

    pltpu.sync_copy(seg_sh, seg_v)

    b_base = w * EPW

    def _gather_wait(b):
        pltpu.make_async_copy(odnum_hbm.at[oidx2.at[b]], rows2.at[b],
                              sem_g.at[b]).wait()

    def _acc_scatter_wait(b):
        pltpu.make_async_copy(rows2.at[b], acc_sh.at[scidx2.at[b]],
                              sem_s.at[b]).wait()

    for b in range(2):
        _idx_load(b, pl.multiple_of(b_base + b * CHUNK, 16), CHUNK)

    def _phase_b(gidx, _):
        for b in range(2):
            k = 2 * gidx + b

            @pl.when(k >= 2)
            def _():
                _acc_scatter_wait(b)

            _idx_wait(b)
            pltpu.async_copy(odnum_hbm.at[oidx2.at[b]], rows2.at[b],
                             sem_g.at[b])
            for j in range(CHUNK // L):
                he = plsc.load_gather(gat_v, [pidx2[b, pl.ds(j * L, L)]])
                ssum = plsc.load_gather(seg_v, [oidx2[b, pl.ds(j * L, L)]])
                ex2[b, pl.ds(j * L, L)] = jnp.exp(he - g) / ssum
            for j in range(CHUNK // L):
                scidx2[b, pl.ds(j * L, L)] = pidx2[b, pl.ds(j * L, L)]
            _gather_wait(b)
            nxt = jnp.minimum(k + 2, B_FULL - 1)
            _idx_load(b, pl.multiple_of(b_base + nxt * CHUNK, 16), CHUNK)

            bsplat = jnp.full((L,), b, jnp.int32)

            @plsc.parallel_loop(0, CHUNK, unroll=4)
            def _(i):
                p = plsc.load_gather(ex2, [bsplat, jnp.full((L,), i, jnp.int32)])
                for j in range(D // L):
                    rows2[b, i, pl.ds(j * L, L)] = (
                        rows2[b, i, pl.ds(j * L, L)] * p)

            pltpu.async_copy(rows2.at[b], acc_sh.at[scidx2.at[b]], sem_s.at[b],
                             add=True)
        return 0

    lax.fori_loop(0, B_FULL // 2, _phase_b, 0)
    for b in range(2):
        _idx_wait(b)
        _acc_scatter_wait(b)

    t_base = pl.multiple_of(b_base + B_FULL * CHUNK, 16)
    pltpu.sync_copy(path_hbm.at[pl.ds(t_base, B_TAIL)], tidxb)
    pltpu.sync_copy(od_hbm.at[pl.ds(t_base, B_TAIL)],
                    oidx2.at[0, pl.ds(0, B_TAIL)])
    pltpu.sync_copy(odnum_hbm.at[oidx2.at[0, pl.ds(0, B_TAIL)]],
                    rows2.at[0, pl.ds(0, B_TAIL), :])
    he = plsc.load_gather(gat_v, [tidxb[pl.ds(0, L)]])
    ssum = plsc.load_gather(seg_v, [oidx2[0, pl.ds(0, L)]])
    ex2[0, pl.ds(0, L)] = jnp.exp(he - g) / ssum

    @plsc.parallel_loop(0, B_TAIL, unroll=2)
    def _(i):
        p = plsc.load_gather(ex2, [jnp.zeros((L,), jnp.int32),
                                   jnp.full((L,), i, jnp.int32)])
        for j in range(D // L):
            rows2[0, i, pl.ds(j * L, L)] = rows2[0, i, pl.ds(j * L, L)] * p

    pltpu.sync_copy(rows2.at[0, pl.ds(0, B_TAIL), :], acc_sh.at[tidxb],
                    add=True)

    plsc.subcore_barrier()
    plsc.subcore_barrier()

    r0 = s * ROWS_PER_TILE
    pltpu.sync_copy(acc_sh.at[pl.ds(r0, ROWS_PER_TILE), :],
                    out_hbm.at[c, pl.ds(r0, ROWS_PER_TILE), :])

    @pl.when(s == 0)
    def _():
        pltpu.sync_copy(acc_sh.at[pl.ds(NS * ROWS_PER_TILE, L), :],
                        out_hbm.at[c, pl.ds(NS * ROWS_PER_TILE, L), :])


@jax.jit
def _sc_call(gat, odnum, path_idx, od_idx):
    mesh = plsc.VectorSubcoreMesh(core_axis_name="c", subcore_axis_name="s")
    kfn = pl.kernel(
        _sc_body,
        mesh=mesh,
        compiler_params=pltpu.CompilerParams(needs_layout_passes=False),
        out_type=jax.ShapeDtypeStruct((NC, N_PATH, D), jnp.float32),
        scratch_types=[
            pltpu.VMEM((N_PATH,), jnp.float32),
            pltpu.VMEM((SEG_PAD,), jnp.float32),
            pltpu.VMEM((2, CHUNK), jnp.int32),
            pltpu.VMEM((2, CHUNK), jnp.int32),
            pltpu.VMEM((2, CHUNK), jnp.int32),
            pltpu.VMEM((2, CHUNK), jnp.float32),
            pltpu.VMEM((2, CHUNK, D), jnp.float32),
            pltpu.VMEM((A_TAIL,), jnp.int32),
            pltpu.VMEM((B_TAIL,), jnp.int32),
            pltpu.SemaphoreType.DMA((2,)),
            pltpu.SemaphoreType.DMA((2,)),
            pltpu.SemaphoreType.DMA((2,)),
            pltpu.VMEM_SHARED((SEG_PAD,), jnp.float32),
            pltpu.VMEM_SHARED((N_PATH, D), jnp.float32),
        ],
    )
    return kfn(gat, odnum, path_idx, od_idx)


def _add_body(p_ref, o_ref):
    o_ref[...] = p_ref[0] + p_ref[1]


@jax.jit
def _combine(partials):
    return pl.pallas_call(
        _add_body,
        out_shape=jax.ShapeDtypeStruct((N_PATH, D), jnp.float32),
        grid=(10,),
        in_specs=[pl.BlockSpec((NC, N_PATH // 10, D), lambda i: (0, i, 0))],
        out_specs=pl.BlockSpec((N_PATH // 10, D), lambda i: (i, 0)),
    )(partials)


def kernel(gatEmb, odNum, path_idx, od_idx):
    gat = jnp.reshape(gatEmb, (N_PATH,))
    partials = _sc_call(gat, odNum, path_idx, od_idx)
    return _combine(partials)


# speedup vs baseline: 45.6223x; 1.4429x over previous
import jax
import jax.numpy as jnp
from jax import lax
from jax.experimental import pallas as pl
from jax.experimental.pallas import tpu as pltpu
from jax.experimental.pallas import tpu_sc as plsc

N_PATH = 10000
N_OD = 10000
E = 320000
D = 128
L = 16
CHUNK = 96
NC = 2
NS = 16
NW = NC * NS
SEG_PAD = 10240
ROWS_PER_TILE = 624
BLK = 4

EPT = E // NS
A_FULL = EPT // CHUNK
A_BLKS = A_FULL // BLK
A_TAIL = EPT - A_FULL * CHUNK
EPW = E // NW
B_FULL = EPW // CHUNK
B_TAIL = EPW - B_FULL * CHUNK


def _sc_body(gat_hbm, odnum_hbm, path_hbm, od_hbm, out_hbm,
             gat_v, seg_v, pidxa, oidxa, scidx4, ex4,
             pidx2, oidx2, scidx2, ex2, rows2,
             tidxa, tidxb, sem_i, sem_g, sem_s, seg_sh, acc_sh):
    c = lax.axis_index("c")
    s = lax.axis_index("s")
    w = s * NC + c

    pltpu.sync_copy(gat_hbm, gat_v)

    def _mx(k, m):
        return jnp.maximum(m, gat_v[pl.ds(k * L, L)])

    m = lax.fori_loop(0, N_PATH // L, _mx,
                      jnp.full((L,), -1e30, dtype=jnp.float32))
    lanes = lax.iota(jnp.int32, L)
    for dstep in (1, 2, 4, 8):
        m = jnp.maximum(m, m.at[lanes ^ dstep].get(mode="promise_in_bounds"))
    g = m

    def _zrows(i, _):
        for j in range(D // L):
            rows2[0, i, pl.ds(j * L, L)] = jnp.zeros((L,), jnp.float32)
        return 0

    lax.fori_loop(0, CHUNK, _zrows, 0)

    for r in range(5):
        pltpu.sync_copy(rows2.at[0, 0, :],
                        seg_sh.at[pl.ds(s * (SEG_PAD // NS) + r * D, D)])
    for r in range(6):
        pltpu.sync_copy(rows2.at[0],
                        acc_sh.at[pl.ds(s * ROWS_PER_TILE + r * CHUNK, CHUNK), :])
    pltpu.sync_copy(rows2.at[0, pl.ds(0, 48), :],
                    acc_sh.at[pl.ds(s * ROWS_PER_TILE + 6 * CHUNK, 48), :])

    @pl.when(s == 0)
    def _():
        pltpu.sync_copy(rows2.at[0, pl.ds(0, L), :],
                        acc_sh.at[pl.ds(NS * ROWS_PER_TILE, L), :])

    plsc.subcore_barrier()
    plsc.subcore_barrier()

    a_base = s * EPT

    def _blk_load(rb, blk):
        base = pl.multiple_of(a_base + blk * (BLK * CHUNK), 32)
        for cb in range(BLK):
            pltpu.async_copy(path_hbm.at[pl.ds(base + cb * CHUNK, CHUNK)],
                             pidxa.at[rb, cb], sem_i.at[rb])
            pltpu.async_copy(od_hbm.at[pl.ds(base + cb * CHUNK, CHUNK)],
                             oidxa.at[rb, cb], sem_i.at[rb])

    def _blk_wait(rb):
        for cb in range(BLK):
            pltpu.make_async_copy(path_hbm.at[pl.ds(0, CHUNK)],
                                  pidxa.at[rb, cb], sem_i.at[rb]).wait()
            pltpu.make_async_copy(od_hbm.at[pl.ds(0, CHUNK)],
                                  oidxa.at[rb, cb], sem_i.at[rb]).wait()

    def _seg_scatter_wait(rb, cb):
        pltpu.make_async_copy(ex4.at[rb, cb], seg_sh.at[scidx4.at[rb, cb]],
                              sem_s.at[rb]).wait()

    for rb in range(2):
        _blk_load(rb, rb)

    def _phase_a(gidx, _):
        for rb in range(2):
            gb = 2 * gidx + rb

            @pl.when(gb >= 2)
            def _():
                for cb in range(BLK):
                    _seg_scatter_wait(rb, cb)

            _blk_wait(rb)
            for cb in range(BLK):
                for j in range(CHUNK // L):
                    he = plsc.load_gather(
                        gat_v, [pidxa[rb, cb, pl.ds(j * L, L)]])
                    ex4[rb, cb, pl.ds(j * L, L)] = jnp.exp(he - g)
                for j in range(CHUNK // L):
                    scidx4[rb, cb, pl.ds(j * L, L)] = (
                        oidxa[rb, cb, pl.ds(j * L, L)])
                pltpu.async_copy(ex4.at[rb, cb], seg_sh.at[scidx4.at[rb, cb]],
                                 sem_s.at[rb], add=True)
            nxt = jnp.minimum(gb + 2, A_BLKS - 1)
            _blk_load(rb, nxt)
        return 0

    lax.fori_loop(0, A_BLKS // 2, _phase_a, 0)
    for rb in range(2):
        _blk_wait(rb)
        for cb in range(BLK):
            _seg_scatter_wait(rb, cb)

    t_base = pl.multiple_of(a_base + A_FULL * CHUNK, 32)
    pltpu.sync_copy(path_hbm.at[pl.ds(t_base, A_TAIL)],
                    pidxa.at[0, 0, pl.ds(0, A_TAIL)])
    pltpu.sync_copy(od_hbm.at[pl.ds(t_base, A_TAIL)], tidxa)
    for j in range(A_TAIL // L):
        he = plsc.load_gather(gat_v, [pidxa[0, 0, pl.ds(j * L, L)]])
        ex4[0, 0, pl.ds(j * L, L)] = jnp.exp(he - g)
    pltpu.sync_copy(ex4.at[0, 0, pl.ds(0, A_TAIL)], seg_sh.at[tidxa], add=True)

    plsc.subcore_barrier()
    plsc.subcore_barrier()

    pltpu.sync_copy(seg_sh, seg_v)

    b_base = w * EPW

    def _pb_idx_load(b, k):
        base = pl.multiple_of(b_base + k * CHUNK, 16)
        pltpu.async_copy(path_hbm.at[pl.ds(base, CHUNK)], pidx2.at[b],
                         sem_i.at[b])
        pltpu.async_copy(od_hbm.at[pl.ds(base, CHUNK)], oidx2.at[b],
                         sem_i.at[b])

    def _pb_idx_wait(b):
        pltpu.make_async_copy(path_hbm.at[pl.ds(0, CHUNK)], pidx2.at[b],
                              sem_i.at[b]).wait()
        pltpu.make_async_copy(od_hbm.at[pl.ds(0, CHUNK)], oidx2.at[b],
                              sem_i.at[b]).wait()

    def _gather_wait(b):
        pltpu.make_async_copy(odnum_hbm.at[oidx2.at[b]], rows2.at[b],
                              sem_g.at[b]).wait()

    def _acc_scatter_wait(b):
        pltpu.make_async_copy(rows2.at[b], acc_sh.at[scidx2.at[b]],
                              sem_s.at[b]).wait()

    def _consume(bp, nxt):
        _gather_wait(bp)
        if nxt is not None:
            _pb_idx_load(bp, nxt)
        bsplat = jnp.full((L,), bp, jnp.int32)

        @plsc.parallel_loop(0, CHUNK, unroll=4)
        def _(i):
            p = plsc.load_gather(ex2, [bsplat, jnp.full((L,), i, jnp.int32)])
            for j in range(D // L):
                rows2[bp, i, pl.ds(j * L, L)] = rows2[bp, i, pl.ds(j * L, L)] * p

        pltpu.async_copy(rows2.at[bp], acc_sh.at[scidx2.at[bp]], sem_s.at[bp],
                         add=True)

    for b in range(2):
        _pb_idx_load(b, b)

    def _phase_b(gidx, _):
        for b in range(2):
            k = 2 * gidx + b
            bp = 1 - b
            _pb_idx_wait(b)

            @pl.when(k >= 2)
            def _():
                _acc_scatter_wait(b)

            pltpu.async_copy(odnum_hbm.at[oidx2.at[b]], rows2.at[b],
                             sem_g.at[b])
            for j in range(CHUNK // L):
                he = plsc.load_gather(gat_v, [pidx2[b, pl.ds(j * L, L)]])
                ssum = plsc.load_gather(seg_v, [oidx2[b, pl.ds(j * L, L)]])
                ex2[b, pl.ds(j * L, L)] = jnp.exp(he - g) / ssum
            for j in range(CHUNK // L):
                scidx2[b, pl.ds(j * L, L)] = pidx2[b, pl.ds(j * L, L)]

            @pl.when(k >= 1)
            def _():
                _consume(bp, jnp.minimum(k + 1, B_FULL - 1))
        return 0

    lax.fori_loop(0, B_FULL // 2, _phase_b, 0)
    _consume(1, None)
    _pb_idx_wait(0)
    for b in range(2):
        _acc_scatter_wait(b)

    t_base = pl.multiple_of(b_base + B_FULL * CHUNK, 16)
    pltpu.sync_copy(path_hbm.at[pl.ds(t_base, B_TAIL)], tidxb)
    pltpu.sync_copy(od_hbm.at[pl.ds(t_base, B_TAIL)],
                    oidx2.at[0, pl.ds(0, B_TAIL)])
    pltpu.sync_copy(odnum_hbm.at[oidx2.at[0, pl.ds(0, B_TAIL)]],
                    rows2.at[0, pl.ds(0, B_TAIL), :])
    he = plsc.load_gather(gat_v, [tidxb[pl.ds(0, L)]])
    ssum = plsc.load_gather(seg_v, [oidx2[0, pl.ds(0, L)]])
    ex2[0, pl.ds(0, L)] = jnp.exp(he - g) / ssum

    @plsc.parallel_loop(0, B_TAIL, unroll=2)
    def _(i):
        p = plsc.load_gather(ex2, [jnp.zeros((L,), jnp.int32),
                                   jnp.full((L,), i, jnp.int32)])
        for j in range(D // L):
            rows2[0, i, pl.ds(j * L, L)] = rows2[0, i, pl.ds(j * L, L)] * p

    pltpu.sync_copy(rows2.at[0, pl.ds(0, B_TAIL), :], acc_sh.at[tidxb],
                    add=True)

    plsc.subcore_barrier()
    plsc.subcore_barrier()

    r0 = s * ROWS_PER_TILE
    pltpu.sync_copy(acc_sh.at[pl.ds(r0, ROWS_PER_TILE), :],
                    out_hbm.at[c, pl.ds(r0, ROWS_PER_TILE), :])

    @pl.when(s == 0)
    def _():
        pltpu.sync_copy(acc_sh.at[pl.ds(NS * ROWS_PER_TILE, L), :],
                        out_hbm.at[c, pl.ds(NS * ROWS_PER_TILE, L), :])


@jax.jit
def _sc_call(gat, odnum, path_idx, od_idx):
    mesh = plsc.VectorSubcoreMesh(core_axis_name="c", subcore_axis_name="s")
    kfn = pl.kernel(
        _sc_body,
        mesh=mesh,
        compiler_params=pltpu.CompilerParams(needs_layout_passes=False),
        out_type=jax.ShapeDtypeStruct((NC, N_PATH, D), jnp.float32),
        scratch_types=[
            pltpu.VMEM((N_PATH,), jnp.float32),
            pltpu.VMEM((SEG_PAD,), jnp.float32),
            pltpu.VMEM((2, BLK, CHUNK), jnp.int32),
            pltpu.VMEM((2, BLK, CHUNK), jnp.int32),
            pltpu.VMEM((2, BLK, CHUNK), jnp.int32),
            pltpu.VMEM((2, BLK, CHUNK), jnp.float32),
            pltpu.VMEM((2, CHUNK), jnp.int32),
            pltpu.VMEM((2, CHUNK), jnp.int32),
            pltpu.VMEM((2, CHUNK), jnp.int32),
            pltpu.VMEM((2, CHUNK), jnp.float32),
            pltpu.VMEM((2, CHUNK, D), jnp.float32),
            pltpu.VMEM((A_TAIL,), jnp.int32),
            pltpu.VMEM((B_TAIL,), jnp.int32),
            pltpu.SemaphoreType.DMA((2,)),
            pltpu.SemaphoreType.DMA((2,)),
            pltpu.SemaphoreType.DMA((2,)),
            pltpu.VMEM_SHARED((SEG_PAD,), jnp.float32),
            pltpu.VMEM_SHARED((N_PATH, D), jnp.float32),
        ],
    )
    return kfn(gat, odnum, path_idx, od_idx)


def _add_body(p_ref, o_ref):
    o_ref[...] = p_ref[0] + p_ref[1]


@jax.jit
def _combine(partials):
    return pl.pallas_call(
        _add_body,
        out_shape=jax.ShapeDtypeStruct((N_PATH, D), jnp.float32),
        grid=(10,),
        in_specs=[pl.BlockSpec((NC, N_PATH // 10, D), lambda i: (0, i, 0))],
        out_specs=pl.BlockSpec((N_PATH // 10, D), lambda i: (i, 0)),
    )(partials)


def kernel(gatEmb, odNum, path_idx, od_idx):
    gat = jnp.reshape(gatEmb, (N_PATH,))
    partials = _sc_call(gat, odNum, path_idx, od_idx)
    return _combine(partials)


# speedup vs baseline: 45.7216x; 1.0022x over previous
import jax
import jax.numpy as jnp
from jax import lax
from jax.experimental import pallas as pl
from jax.experimental.pallas import tpu as pltpu
from jax.experimental.pallas import tpu_sc as plsc

N_PATH = 10000
N_OD = 10000
E = 320000
D = 128
L = 16
CHUNK = 96
NC = 2
NS = 16
NW = NC * NS
SEG_PAD = 10240
ROWS_PER_TILE = 624
BLK = 4

EPT = E // NS
A_FULL = EPT // CHUNK
A_BLKS = A_FULL // BLK
A_TAIL = EPT - A_FULL * CHUNK
EPW = E // NW
B_FULL = EPW // CHUNK
B_TAIL = EPW - B_FULL * CHUNK


def _sc_body(gat_hbm, odnum_hbm, path_hbm, od_hbm, out_hbm,
             gat_v, seg_v, pidxa, oidxa, scidx4, ex4,
             pidx2, oidx2, scidx2, ex2, rows2,
             tidxa, tidxb, sem_i, sem_g, sem_s, seg_sh, acc_sh):
    c = lax.axis_index("c")
    s = lax.axis_index("s")
    w = s * NC + c

    pltpu.sync_copy(gat_hbm, gat_v)

    def _mx(k, m):
        return jnp.maximum(m, gat_v[pl.ds(k * L, L)])

    m = lax.fori_loop(0, N_PATH // L, _mx,
                      jnp.full((L,), -1e30, dtype=jnp.float32))
    lanes = lax.iota(jnp.int32, L)
    for dstep in (1, 2, 4, 8):
        m = jnp.maximum(m, m.at[lanes ^ dstep].get(mode="promise_in_bounds"))
    g = m

    def _zrows(i, _):
        for j in range(D // L):
            rows2[0, i, pl.ds(j * L, L)] = jnp.zeros((L,), jnp.float32)
        return 0

    lax.fori_loop(0, CHUNK, _zrows, 0)

    for r in range(5):
        pltpu.sync_copy(rows2.at[0, 0, :],
                        seg_sh.at[pl.ds(s * (SEG_PAD // NS) + r * D, D)])
    for r in range(6):
        pltpu.sync_copy(rows2.at[0],
                        acc_sh.at[pl.ds(s * ROWS_PER_TILE + r * CHUNK, CHUNK), :])
    pltpu.sync_copy(rows2.at[0, pl.ds(0, 48), :],
                    acc_sh.at[pl.ds(s * ROWS_PER_TILE + 6 * CHUNK, 48), :])

    @pl.when(s == 0)
    def _():
        pltpu.sync_copy(rows2.at[0, pl.ds(0, L), :],
                        acc_sh.at[pl.ds(NS * ROWS_PER_TILE, L), :])

    plsc.subcore_barrier()
    plsc.subcore_barrier()

    a_base = s * EPT

    def _blk_load(rb, blk):
        base = pl.multiple_of(a_base + blk * (BLK * CHUNK), 32)
        for cb in range(BLK):
            pltpu.async_copy(path_hbm.at[pl.ds(base + cb * CHUNK, CHUNK)],
                             pidxa.at[rb, cb], sem_i.at[rb])
            pltpu.async_copy(od_hbm.at[pl.ds(base + cb * CHUNK, CHUNK)],
                             oidxa.at[rb, cb], sem_i.at[rb])

    def _blk_wait(rb):
        for cb in range(BLK):
            pltpu.make_async_copy(path_hbm.at[pl.ds(0, CHUNK)],
                                  pidxa.at[rb, cb], sem_i.at[rb]).wait()
            pltpu.make_async_copy(od_hbm.at[pl.ds(0, CHUNK)],
                                  oidxa.at[rb, cb], sem_i.at[rb]).wait()

    def _seg_scatter_wait(rb, cb):
        pltpu.make_async_copy(ex4.at[rb, cb], seg_sh.at[scidx4.at[rb, cb]],
                              sem_s.at[rb]).wait()

    for rb in range(2):
        _blk_load(rb, rb)

    def _phase_a(gidx, _):
        for rb in range(2):
            gb = 2 * gidx + rb

            @pl.when(gb >= 2)
            def _():
                for cb in range(BLK):
                    _seg_scatter_wait(rb, cb)

            _blk_wait(rb)
            for cb in range(BLK):
                for j in range(CHUNK // L):
                    he = plsc.load_gather(
                        gat_v, [pidxa[rb, cb, pl.ds(j * L, L)]])
                    ex4[rb, cb, pl.ds(j * L, L)] = jnp.exp(he - g)
                for j in range(CHUNK // L):
                    scidx4[rb, cb, pl.ds(j * L, L)] = (
                        oidxa[rb, cb, pl.ds(j * L, L)])
                pltpu.async_copy(ex4.at[rb, cb], seg_sh.at[scidx4.at[rb, cb]],
                                 sem_s.at[rb], add=True)
            nxt = jnp.minimum(gb + 2, A_BLKS - 1)
            _blk_load(rb, nxt)
        return 0

    lax.fori_loop(0, A_BLKS // 2, _phase_a, 0)
    for rb in range(2):
        _blk_wait(rb)
        for cb in range(BLK):
            _seg_scatter_wait(rb, cb)

    t_base = pl.multiple_of(a_base + A_FULL * CHUNK, 32)
    pltpu.sync_copy(path_hbm.at[pl.ds(t_base, A_TAIL)],
                    pidxa.at[0, 0, pl.ds(0, A_TAIL)])
    pltpu.sync_copy(od_hbm.at[pl.ds(t_base, A_TAIL)], tidxa)
    for j in range(A_TAIL // L):
        he = plsc.load_gather(gat_v, [pidxa[0, 0, pl.ds(j * L, L)]])
        ex4[0, 0, pl.ds(j * L, L)] = jnp.exp(he - g)
    pltpu.sync_copy(ex4.at[0, 0, pl.ds(0, A_TAIL)], seg_sh.at[tidxa], add=True)

    plsc.subcore_barrier()
    plsc.subcore_barrier()

    pltpu.sync_copy(seg_sh, seg_v)

    b_base = w * EPW

    def _pb_idx_load(b, k):
        base = pl.multiple_of(b_base + k * CHUNK, 16)
        pltpu.async_copy(path_hbm.at[pl.ds(base, CHUNK)], pidx2.at[b],
                         sem_i.at[b])
        pltpu.async_copy(od_hbm.at[pl.ds(base, CHUNK)], oidx2.at[b],
                         sem_i.at[b])

    def _pb_idx_wait(b):
        pltpu.make_async_copy(path_hbm.at[pl.ds(0, CHUNK)], pidx2.at[b],
                              sem_i.at[b]).wait()
        pltpu.make_async_copy(od_hbm.at[pl.ds(0, CHUNK)], oidx2.at[b],
                              sem_i.at[b]).wait()

    def _gather_wait(b):
        pltpu.make_async_copy(odnum_hbm.at[oidx2.at[b]], rows2.at[b],
                              sem_g.at[b]).wait()

    def _acc_scatter_wait(b):
        pltpu.make_async_copy(rows2.at[b], acc_sh.at[scidx2.at[b]],
                              sem_s.at[b]).wait()

    def _consume(bp, nxt):
        _gather_wait(bp)
        if nxt is not None:
            _pb_idx_load(bp, nxt)

        @plsc.parallel_loop(0, CHUNK // L, unroll=2)
        def _(t):
            exv = ex2[bp, pl.ds(t * L, L)]
            for u in range(L):
                p = exv.at[jnp.full((L,), u, jnp.int32)].get(
                    mode="promise_in_bounds")
                r = t * L + u
                for j in range(D // L):
                    rows2[bp, r, pl.ds(j * L, L)] = (
                        rows2[bp, r, pl.ds(j * L, L)] * p)

        pltpu.async_copy(rows2.at[bp], acc_sh.at[scidx2.at[bp]], sem_s.at[bp],
                         add=True)

    for b in range(2):
        _pb_idx_load(b, b)

    def _phase_b(gidx, _):
        for b in range(2):
            k = 2 * gidx + b
            bp = 1 - b
            _pb_idx_wait(b)

            @pl.when(k >= 2)
            def _():
                _acc_scatter_wait(b)

            pltpu.async_copy(odnum_hbm.at[oidx2.at[b]], rows2.at[b],
                             sem_g.at[b])
            for j in range(CHUNK // L):
                he = plsc.load_gather(gat_v, [pidx2[b, pl.ds(j * L, L)]])
                ssum = plsc.load_gather(seg_v, [oidx2[b, pl.ds(j * L, L)]])
                ex2[b, pl.ds(j * L, L)] = jnp.exp(he - g) / ssum
            for j in range(CHUNK // L):
                scidx2[b, pl.ds(j * L, L)] = pidx2[b, pl.ds(j * L, L)]

            @pl.when(k >= 1)
            def _():
                _consume(bp, jnp.minimum(k + 1, B_FULL - 1))
        return 0

    lax.fori_loop(0, B_FULL // 2, _phase_b, 0)
    _consume(1, None)
    _pb_idx_wait(0)
    for b in range(2):
        _acc_scatter_wait(b)

    t_base = pl.multiple_of(b_base + B_FULL * CHUNK, 16)
    pltpu.sync_copy(path_hbm.at[pl.ds(t_base, B_TAIL)], tidxb)
    pltpu.sync_copy(od_hbm.at[pl.ds(t_base, B_TAIL)],
                    oidx2.at[0, pl.ds(0, B_TAIL)])
    pltpu.sync_copy(odnum_hbm.at[oidx2.at[0, pl.ds(0, B_TAIL)]],
                    rows2.at[0, pl.ds(0, B_TAIL), :])
    he = plsc.load_gather(gat_v, [tidxb[pl.ds(0, L)]])
    ssum = plsc.load_gather(seg_v, [oidx2[0, pl.ds(0, L)]])
    ex2[0, pl.ds(0, L)] = jnp.exp(he - g) / ssum

    @plsc.parallel_loop(0, B_TAIL, unroll=2)
    def _(i):
        p = plsc.load_gather(ex2, [jnp.zeros((L,), jnp.int32),
                                   jnp.full((L,), i, jnp.int32)])
        for j in range(D // L):
            rows2[0, i, pl.ds(j * L, L)] = rows2[0, i, pl.ds(j * L, L)] * p

    pltpu.sync_copy(rows2.at[0, pl.ds(0, B_TAIL), :], acc_sh.at[tidxb],
                    add=True)

    plsc.subcore_barrier()
    plsc.subcore_barrier()

    r0 = s * ROWS_PER_TILE
    pltpu.sync_copy(acc_sh.at[pl.ds(r0, ROWS_PER_TILE), :],
                    out_hbm.at[c, pl.ds(r0, ROWS_PER_TILE), :])

    @pl.when(s == 0)
    def _():
        pltpu.sync_copy(acc_sh.at[pl.ds(NS * ROWS_PER_TILE, L), :],
                        out_hbm.at[c, pl.ds(NS * ROWS_PER_TILE, L), :])


@jax.jit
def _sc_call(gat, odnum, path_idx, od_idx):
    mesh = plsc.VectorSubcoreMesh(core_axis_name="c", subcore_axis_name="s")
    kfn = pl.kernel(
        _sc_body,
        mesh=mesh,
        compiler_params=pltpu.CompilerParams(needs_layout_passes=False),
        out_type=jax.ShapeDtypeStruct((NC, N_PATH, D), jnp.float32),
        scratch_types=[
            pltpu.VMEM((N_PATH,), jnp.float32),
            pltpu.VMEM((SEG_PAD,), jnp.float32),
            pltpu.VMEM((2, BLK, CHUNK), jnp.int32),
            pltpu.VMEM((2, BLK, CHUNK), jnp.int32),
            pltpu.VMEM((2, BLK, CHUNK), jnp.int32),
            pltpu.VMEM((2, BLK, CHUNK), jnp.float32),
            pltpu.VMEM((2, CHUNK), jnp.int32),
            pltpu.VMEM((2, CHUNK), jnp.int32),
            pltpu.VMEM((2, CHUNK), jnp.int32),
            pltpu.VMEM((2, CHUNK), jnp.float32),
            pltpu.VMEM((2, CHUNK, D), jnp.float32),
            pltpu.VMEM((A_TAIL,), jnp.int32),
            pltpu.VMEM((B_TAIL,), jnp.int32),
            pltpu.SemaphoreType.DMA((2,)),
            pltpu.SemaphoreType.DMA((2,)),
            pltpu.SemaphoreType.DMA((2,)),
            pltpu.VMEM_SHARED((SEG_PAD,), jnp.float32),
            pltpu.VMEM_SHARED((N_PATH, D), jnp.float32),
        ],
    )
    return kfn(gat, odnum, path_idx, od_idx)


def _add_body(p_ref, o_ref):
    o_ref[...] = p_ref[0] + p_ref[1]


@jax.jit
def _combine(partials):
    return pl.pallas_call(
        _add_body,
        out_shape=jax.ShapeDtypeStruct((N_PATH, D), jnp.float32),
        grid=(10,),
        in_specs=[pl.BlockSpec((NC, N_PATH // 10, D), lambda i: (0, i, 0))],
        out_specs=pl.BlockSpec((N_PATH // 10, D), lambda i: (i, 0)),
    )(partials)


def kernel(gatEmb, odNum, path_idx, od_idx):
    gat = jnp.reshape(gatEmb, (N_PATH,))
    partials = _sc_call(gat, odNum, path_idx, od_idx)
    return _combine(partials)


# speedup vs baseline: 45.8069x; 1.0019x over previous
import jax
import jax.numpy as jnp
from jax import lax
from jax.experimental import pallas as pl
from jax.experimental.pallas import tpu as pltpu
from jax.experimental.pallas import tpu_sc as plsc

N_PATH = 10000
N_OD = 10000
E = 320000
D = 128
L = 16
CHUNK = 96
NC = 2
NS = 16
NW = NC * NS
SEG_PAD = 10240
ROWS_PER_TILE = 624
BLK = 4

EPT = E // NS
A_FULL = EPT // CHUNK
A_BLKS = A_FULL // BLK
A_TAIL = EPT - A_FULL * CHUNK
EPW = E // NW
B_FULL = EPW // CHUNK
B_TAIL = EPW - B_FULL * CHUNK


def _sc_body(gat_hbm, odnum_hbm, path_hbm, od_hbm, out_hbm,
             gat_v, seg_v, pidxa, oidxa, scidx4, ex4,
             pidx2, oidx2, scidx2, ex2, rows2,
             tidxa, tidxb, sem_i, sem_g, sem_s, seg_sh, acc_sh):
    c = lax.axis_index("c")
    s = lax.axis_index("s")
    w = s * NC + c

    pltpu.sync_copy(gat_hbm, gat_v)

    def _mx(k, m):
        return jnp.maximum(m, gat_v[pl.ds(k * L, L)])

    m = lax.fori_loop(0, N_PATH // L, _mx,
                      jnp.full((L,), -1e30, dtype=jnp.float32))
    lanes = lax.iota(jnp.int32, L)
    for dstep in (1, 2, 4, 8):
        m = jnp.maximum(m, m.at[lanes ^ dstep].get(mode="promise_in_bounds"))
    g = m

    def _zrows(i, _):
        for j in range(D // L):
            rows2[0, i, pl.ds(j * L, L)] = jnp.zeros((L,), jnp.float32)
        return 0

    lax.fori_loop(0, CHUNK, _zrows, 0)

    for r in range(5):
        pltpu.sync_copy(rows2.at[0, 0, :],
                        seg_sh.at[pl.ds(s * (SEG_PAD // NS) + r * D, D)])
    for r in range(6):
        pltpu.sync_copy(rows2.at[0],
                        acc_sh.at[pl.ds(s * ROWS_PER_TILE + r * CHUNK, CHUNK), :])
    pltpu.sync_copy(rows2.at[0, pl.ds(0, 48), :],
                    acc_sh.at[pl.ds(s * ROWS_PER_TILE + 6 * CHUNK, 48), :])

    @pl.when(s == 0)
    def _():
        pltpu.sync_copy(rows2.at[0, pl.ds(0, L), :],
                        acc_sh.at[pl.ds(NS * ROWS_PER_TILE, L), :])

    plsc.subcore_barrier()
    plsc.subcore_barrier()

    a_base = s * EPT

    def _blk_load(rb, blk):
        base = pl.multiple_of(a_base + blk * (BLK * CHUNK), 32)
        for cb in range(BLK):
            pltpu.async_copy(path_hbm.at[pl.ds(base + cb * CHUNK, CHUNK)],
                             pidxa.at[rb, cb], sem_i.at[rb])
            pltpu.async_copy(od_hbm.at[pl.ds(base + cb * CHUNK, CHUNK)],
                             oidxa.at[rb, cb], sem_i.at[rb])

    def _blk_wait(rb):
        for cb in range(BLK):
            pltpu.make_async_copy(path_hbm.at[pl.ds(0, CHUNK)],
                                  pidxa.at[rb, cb], sem_i.at[rb]).wait()
            pltpu.make_async_copy(od_hbm.at[pl.ds(0, CHUNK)],
                                  oidxa.at[rb, cb], sem_i.at[rb]).wait()

    def _seg_scatter_wait(rb, cb):
        pltpu.make_async_copy(ex4.at[rb, cb], seg_sh.at[scidx4.at[rb, cb]],
                              sem_s.at[rb]).wait()

    for rb in range(2):
        _blk_load(rb, rb)

    def _phase_a(gidx, _):
        for rb in range(2):
            gb = 2 * gidx + rb

            @pl.when(gb >= 2)
            def _():
                for cb in range(BLK):
                    _seg_scatter_wait(rb, cb)

            _blk_wait(rb)
            for cb in range(BLK):
                for j in range(CHUNK // L):
                    he = plsc.load_gather(
                        gat_v, [pidxa[rb, cb, pl.ds(j * L, L)]])
                    ex4[rb, cb, pl.ds(j * L, L)] = jnp.exp(he - g)
                for j in range(CHUNK // L):
                    scidx4[rb, cb, pl.ds(j * L, L)] = (
                        oidxa[rb, cb, pl.ds(j * L, L)])
                pltpu.async_copy(ex4.at[rb, cb], seg_sh.at[scidx4.at[rb, cb]],
                                 sem_s.at[rb], add=True)
            nxt = jnp.minimum(gb + 2, A_BLKS - 1)
            _blk_load(rb, nxt)
        return 0

    lax.fori_loop(0, A_BLKS // 2, _phase_a, 0)
    for rb in range(2):
        _blk_wait(rb)
        for cb in range(BLK):
            _seg_scatter_wait(rb, cb)

    t_base = pl.multiple_of(a_base + A_FULL * CHUNK, 32)
    pltpu.sync_copy(path_hbm.at[pl.ds(t_base, A_TAIL)],
                    pidxa.at[0, 0, pl.ds(0, A_TAIL)])
    pltpu.sync_copy(od_hbm.at[pl.ds(t_base, A_TAIL)], tidxa)
    for j in range(A_TAIL // L):
        he = plsc.load_gather(gat_v, [pidxa[0, 0, pl.ds(j * L, L)]])
        ex4[0, 0, pl.ds(j * L, L)] = jnp.exp(he - g)
    pltpu.sync_copy(ex4.at[0, 0, pl.ds(0, A_TAIL)], seg_sh.at[tidxa], add=True)

    plsc.subcore_barrier()
    plsc.subcore_barrier()

    pltpu.sync_copy(seg_sh, seg_v)

    b_base = w * EPW

    def _pb_idx_load(b, k):
        base = pl.multiple_of(b_base + k * CHUNK, 16)
        pltpu.async_copy(path_hbm.at[pl.ds(base, CHUNK)], pidx2.at[b],
                         sem_i.at[b])
        pltpu.async_copy(od_hbm.at[pl.ds(base, CHUNK)], oidx2.at[b],
                         sem_i.at[b])

    def _pb_idx_wait(b):
        pltpu.make_async_copy(path_hbm.at[pl.ds(0, CHUNK)], pidx2.at[b],
                              sem_i.at[b]).wait()
        pltpu.make_async_copy(od_hbm.at[pl.ds(0, CHUNK)], oidx2.at[b],
                              sem_i.at[b]).wait()

    def _gather_wait(b):
        pltpu.make_async_copy(odnum_hbm.at[oidx2.at[b]], rows2.at[b],
                              sem_g.at[b]).wait()

    def _acc_scatter_wait(b):
        pltpu.make_async_copy(rows2.at[b], acc_sh.at[scidx2.at[b]],
                              sem_s.at[b]).wait()

    def _consume(bp, nxt):
        _gather_wait(bp)
        if nxt is not None:
            _pb_idx_load(bp, nxt)

        @plsc.parallel_loop(0, CHUNK // L, unroll=2)
        def _(t):
            exv = ex2[bp, pl.ds(t * L, L)]
            for u in range(L):
                p = exv.at[jnp.full((L,), u, jnp.int32)].get(
                    mode="promise_in_bounds")
                r = t * L + u
                for j in range(D // L):
                    rows2[bp, r, pl.ds(j * L, L)] = (
                        rows2[bp, r, pl.ds(j * L, L)] * p)

        pltpu.async_copy(rows2.at[bp], acc_sh.at[scidx2.at[bp]], sem_s.at[bp],
                         add=True)

    for b in range(2):
        _pb_idx_load(b, b)

    def _phase_b(gidx, _):
        for b in range(2):
            k = 2 * gidx + b
            bp = 1 - b
            _pb_idx_wait(b)

            @pl.when(k >= 2)
            def _():
                _acc_scatter_wait(b)

            pltpu.async_copy(odnum_hbm.at[oidx2.at[b]], rows2.at[b],
                             sem_g.at[b])
            for j in range(CHUNK // L):
                he = plsc.load_gather(gat_v, [pidx2[b, pl.ds(j * L, L)]])
                ssum = plsc.load_gather(seg_v, [oidx2[b, pl.ds(j * L, L)]])
                ex2[b, pl.ds(j * L, L)] = jnp.exp(he - g) / ssum
            for j in range(CHUNK // L):
                scidx2[b, pl.ds(j * L, L)] = pidx2[b, pl.ds(j * L, L)]

            @pl.when(k >= 1)
            def _():
                _consume(bp, jnp.minimum(k + 1, B_FULL - 1))
        return 0

    lax.fori_loop(0, B_FULL // 2, _phase_b, 0)
    _consume(1, None)
    _pb_idx_wait(0)
    for b in range(2):
        _acc_scatter_wait(b)

    t_base = pl.multiple_of(b_base + B_FULL * CHUNK, 16)
    pltpu.sync_copy(path_hbm.at[pl.ds(t_base, B_TAIL)], tidxb)
    pltpu.sync_copy(od_hbm.at[pl.ds(t_base, B_TAIL)],
                    oidx2.at[0, pl.ds(0, B_TAIL)])
    pltpu.sync_copy(odnum_hbm.at[oidx2.at[0, pl.ds(0, B_TAIL)]],
                    rows2.at[0, pl.ds(0, B_TAIL), :])
    he = plsc.load_gather(gat_v, [tidxb[pl.ds(0, L)]])
    ssum = plsc.load_gather(seg_v, [oidx2[0, pl.ds(0, L)]])
    ex2[0, pl.ds(0, L)] = jnp.exp(he - g) / ssum

    @plsc.parallel_loop(0, B_TAIL, unroll=2)
    def _(i):
        p = plsc.load_gather(ex2, [jnp.zeros((L,), jnp.int32),
                                   jnp.full((L,), i, jnp.int32)])
        for j in range(D // L):
            rows2[0, i, pl.ds(j * L, L)] = rows2[0, i, pl.ds(j * L, L)] * p

    pltpu.sync_copy(rows2.at[0, pl.ds(0, B_TAIL), :], acc_sh.at[tidxb],
                    add=True)

    plsc.subcore_barrier()
    plsc.subcore_barrier()

    r0 = s * ROWS_PER_TILE
    pltpu.sync_copy(acc_sh.at[pl.ds(r0, ROWS_PER_TILE), :],
                    out_hbm.at[c, pl.ds(r0, ROWS_PER_TILE), :])

    @pl.when(s == 0)
    def _():
        pltpu.sync_copy(acc_sh.at[pl.ds(NS * ROWS_PER_TILE, L), :],
                        out_hbm.at[c, pl.ds(NS * ROWS_PER_TILE, L), :])


def _sc_call(gat, odnum, path_idx, od_idx):
    mesh = plsc.VectorSubcoreMesh(core_axis_name="c", subcore_axis_name="s")
    kfn = pl.kernel(
        _sc_body,
        mesh=mesh,
        compiler_params=pltpu.CompilerParams(needs_layout_passes=False),
        out_type=jax.ShapeDtypeStruct((NC, N_PATH, D), jnp.float32),
        scratch_types=[
            pltpu.VMEM((N_PATH,), jnp.float32),
            pltpu.VMEM((SEG_PAD,), jnp.float32),
            pltpu.VMEM((2, BLK, CHUNK), jnp.int32),
            pltpu.VMEM((2, BLK, CHUNK), jnp.int32),
            pltpu.VMEM((2, BLK, CHUNK), jnp.int32),
            pltpu.VMEM((2, BLK, CHUNK), jnp.float32),
            pltpu.VMEM((2, CHUNK), jnp.int32),
            pltpu.VMEM((2, CHUNK), jnp.int32),
            pltpu.VMEM((2, CHUNK), jnp.int32),
            pltpu.VMEM((2, CHUNK), jnp.float32),
            pltpu.VMEM((2, CHUNK, D), jnp.float32),
            pltpu.VMEM((A_TAIL,), jnp.int32),
            pltpu.VMEM((B_TAIL,), jnp.int32),
            pltpu.SemaphoreType.DMA((2,)),
            pltpu.SemaphoreType.DMA((2,)),
            pltpu.SemaphoreType.DMA((2,)),
            pltpu.VMEM_SHARED((SEG_PAD,), jnp.float32),
            pltpu.VMEM_SHARED((N_PATH, D), jnp.float32),
        ],
    )
    return kfn(gat, odnum, path_idx, od_idx)


def _add_body(p_ref, o_ref):
    o_ref[...] = p_ref[0] + p_ref[1]


def _combine(partials):
    return pl.pallas_call(
        _add_body,
        out_shape=jax.ShapeDtypeStruct((N_PATH, D), jnp.float32),
        grid=(10,),
        in_specs=[pl.BlockSpec((NC, N_PATH // 10, D), lambda i: (0, i, 0))],
        out_specs=pl.BlockSpec((N_PATH // 10, D), lambda i: (i, 0)),
    )(partials)


@jax.jit
def kernel(gatEmb, odNum, path_idx, od_idx):
    gat = jnp.reshape(gatEmb, (N_PATH,))
    partials = _sc_call(gat, odNum, path_idx, od_idx)
    return _combine(partials)
